# baseline (speedup 1.0000x reference)
import functools

import jax
import jax.numpy as jnp
from jax import lax
from jax.experimental import pallas as pl
from jax.experimental.pallas import tpu as pltpu
from jax.experimental.pallas import tpu_sc as plsc

N_NODES = 248
N_EDGES = 4096
_NPAD = 256
_MFLAT = N_NODES * N_NODES
_ROWS = N_EDGES // 128
_LUT = 4104


def _sc_body(edges_hbm, lut_hbm, zeros_hbm, ones_hbm, out_hbm,
             edges_v, lut_v, ones_v, deg_v, dinv_v, norm_v, fidx_v,
             m_sh, deg_sh, sem):
    cid = lax.axis_index("c")
    sid = lax.axis_index("s")

    @pl.when(jnp.logical_and(cid == 0, sid == 0))
    def _():
        pltpu.sync_copy(edges_hbm, edges_v)
        pltpu.sync_copy(lut_hbm, lut_v)
        pltpu.sync_copy(ones_hbm, ones_v)
        pltpu.sync_copy(zeros_hbm, m_sh)
        pltpu.sync_copy(zeros_hbm.at[pl.ds(0, _NPAD)], deg_sh)

        handles = []
        for j in range(_ROWS):
            handles.append(
                pltpu.async_copy(ones_v.at[j], deg_sh.at[edges_v.at[_ROWS + j]],
                                 sem, add=True))
        for h in handles:
            h.wait()

        pltpu.sync_copy(deg_sh, deg_v)
        for i in range(_NPAD // 16):
            cnt = deg_v[pl.ds(i * 16, 16)].astype(jnp.int32)
            dinv_v[pl.ds(i * 16, 16)] = plsc.load_gather(lut_v, [cnt])

        for j in range(_ROWS):
            for k in range(8):
                s16 = edges_v[j, pl.ds(k * 16, 16)]
                d16 = edges_v[_ROWS + j, pl.ds(k * 16, 16)]
                ns = plsc.load_gather(dinv_v, [s16])
                nd = plsc.load_gather(dinv_v, [d16])
                norm_v[j, pl.ds(k * 16, 16)] = ns * nd
                fidx_v[j, pl.ds(k * 16, 16)] = s16 * N_NODES + d16

        for i in range(_NPAD // 16):
            jvec = lax.iota(jnp.int32, 16) + i * 16
            valid = jvec < N_NODES
            dv = dinv_v[pl.ds(i * 16, 16)]
            row = _ROWS + i // 8
            col = (i % 8) * 16
            norm_v[row, pl.ds(col, 16)] = jnp.where(valid, dv * dv, 0.0)
            fidx_v[row, pl.ds(col, 16)] = jnp.where(
                valid, jvec * (N_NODES + 1), _MFLAT - 1)

        handles = []
        for j in range(_ROWS + 2):
            handles.append(
                pltpu.async_copy(norm_v.at[j], m_sh.at[fidx_v.at[j]],
                                 sem, add=True))
        for h in handles:
            h.wait()

        pltpu.sync_copy(m_sh, out_hbm)


@jax.jit
def _build_operator(edge_index):
    edges2 = edge_index.reshape(2 * _ROWS, 128)
    lut = lax.rsqrt(jnp.arange(1, _LUT + 1, dtype=jnp.float32))
    zeros = jnp.zeros((_MFLAT,), jnp.float32)
    ones = jnp.ones((_ROWS, 128), jnp.float32)
    mesh = plsc.VectorSubcoreMesh(core_axis_name="c", subcore_axis_name="s")
    m_flat = pl.kernel(
        _sc_body,
        out_type=jax.ShapeDtypeStruct((_MFLAT,), jnp.float32),
        mesh=mesh,
        compiler_params=pltpu.CompilerParams(needs_layout_passes=False),
        scratch_types=[
            pltpu.VMEM((2 * _ROWS, 128), jnp.int32),
            pltpu.VMEM((_LUT,), jnp.float32),
            pltpu.VMEM((_ROWS, 128), jnp.float32),
            pltpu.VMEM((_NPAD,), jnp.float32),
            pltpu.VMEM((_NPAD,), jnp.float32),
            pltpu.VMEM((_ROWS + 2, 128), jnp.float32),
            pltpu.VMEM((_ROWS + 2, 128), jnp.int32),
            pltpu.VMEM_SHARED((_MFLAT,), jnp.float32),
            pltpu.VMEM_SHARED((_NPAD,), jnp.float32),
            pltpu.SemaphoreType.DMA,
        ],
    )(edges2, lut, zeros, ones)
    return m_flat.reshape(N_NODES, N_NODES)


def _tc_body(params_ref, m_ref, w2_ref, b2_ref, w3_ref, b3_ref, x_ref,
             out_ref, xcopy_ref):
    b1 = params_ref[1]
    s2 = params_ref[2]
    bb = params_ref[3]
    xv = x_ref[...]
    xcopy_ref[:, 0, :] = xv
    h = lax.dot_general(xv, m_ref[...], (((1,), (0,)), ((), ())),
                        preferred_element_type=jnp.float32)
    h = jnp.tanh(h + b1) * s2 + bb
    h = lax.dot_general(h, w2_ref[...], (((1,), (1,)), ((), ())),
                        preferred_element_type=jnp.float32) + b2_ref[...]
    h = jnp.tanh(h)
    out_ref[:, 0, :] = lax.dot_general(h, w3_ref[...], (((1,), (1,)), ((), ())),
                                       preferred_element_type=jnp.float32) + b3_ref[...]


def _fused_mlp(params, m, w2, b2, w3, b3, x2, block_b=1024):
    batch = x2.shape[0]
    n = x2.shape[1]
    d2 = w2.shape[0]
    d3 = w3.shape[0]
    grid = (batch // block_b,)
    return pl.pallas_call(
        _tc_body,
        grid=grid,
        in_specs=[
            pl.BlockSpec(memory_space=pltpu.SMEM),
            pl.BlockSpec((n, n), lambda i: (0, 0)),
            pl.BlockSpec((d2, n), lambda i: (0, 0)),
            pl.BlockSpec((1, d2), lambda i: (0, 0)),
            pl.BlockSpec((d3, d2), lambda i: (0, 0)),
            pl.BlockSpec((1, d3), lambda i: (0, 0)),
            pl.BlockSpec((block_b, n), lambda i: (i, 0)),
        ],
        out_specs=[pl.BlockSpec((block_b, 1, d3), lambda i: (i, 0, 0)),
                   pl.BlockSpec((block_b, 1, n), lambda i: (i, 0, 0))],
        out_shape=[jax.ShapeDtypeStruct((batch, 1, d3), jnp.float32),
                   jax.ShapeDtypeStruct((batch, 1, n), jnp.float32)],
        compiler_params=pltpu.CompilerParams(
            dimension_semantics=("arbitrary",)),
    )(params, m, w2, b2, w3, b3, x2)


def kernel(x, edge_index, gcn_w, gcn_b, bn_gamma, bn_beta, w2, b2, w3, b3):
    batch = x.shape[0]
    m = _build_operator(edge_index) * gcn_w[0, 0]
    params = jnp.stack([
        jnp.float32(1.0),
        gcn_b[0],
        bn_gamma[0] * lax.rsqrt(jnp.float32(1.0 + 1e-5)),
        bn_beta[0],
    ])
    x2 = x.reshape(batch, N_NODES)
    out, xcopy = _fused_mlp(params, m, w2, b2.reshape(1, -1), w3,
                            b3.reshape(1, -1), x2)
    return (xcopy, out)


# speedup vs baseline: 1.5626x; 1.5626x over previous
import functools

import jax
import jax.numpy as jnp
from jax import lax
from jax.experimental import pallas as pl
from jax.experimental.pallas import tpu as pltpu
from jax.experimental.pallas import tpu_sc as plsc

N_NODES = 248
N_EDGES = 4096
_NPAD = 256
_MFLAT = N_NODES * N_NODES
_ROWS = N_EDGES // 128
_LUT = 4104


def _sc_body(edges_hbm, lut_hbm, zeros_hbm, ones_hbm, out_hbm,
             edges_v, lut_v, ones_v, deg_v, dinv_v, norm_v, fidx_v,
             m_sh, deg_sh, sem):
    cid = lax.axis_index("c")
    sid = lax.axis_index("s")

    @pl.when(jnp.logical_and(cid == 0, sid == 0))
    def _():
        pltpu.sync_copy(edges_hbm, edges_v)
        pltpu.sync_copy(lut_hbm, lut_v)
        pltpu.sync_copy(ones_hbm, ones_v)
        pltpu.sync_copy(zeros_hbm, m_sh)
        pltpu.sync_copy(zeros_hbm.at[pl.ds(0, _NPAD)], deg_sh)

        handles = []
        for j in range(_ROWS):
            handles.append(
                pltpu.async_copy(ones_v.at[j], deg_sh.at[edges_v.at[_ROWS + j]],
                                 sem, add=True))
        for h in handles:
            h.wait()

        pltpu.sync_copy(deg_sh, deg_v)
        for i in range(_NPAD // 16):
            cnt = deg_v[pl.ds(i * 16, 16)].astype(jnp.int32)
            dinv_v[pl.ds(i * 16, 16)] = plsc.load_gather(lut_v, [cnt])

        for j in range(_ROWS):
            for k in range(8):
                s16 = edges_v[j, pl.ds(k * 16, 16)]
                d16 = edges_v[_ROWS + j, pl.ds(k * 16, 16)]
                ns = plsc.load_gather(dinv_v, [s16])
                nd = plsc.load_gather(dinv_v, [d16])
                norm_v[j, pl.ds(k * 16, 16)] = ns * nd
                fidx_v[j, pl.ds(k * 16, 16)] = s16 * N_NODES + d16

        for i in range(_NPAD // 16):
            jvec = lax.iota(jnp.int32, 16) + i * 16
            valid = jvec < N_NODES
            dv = dinv_v[pl.ds(i * 16, 16)]
            row = _ROWS + i // 8
            col = (i % 8) * 16
            norm_v[row, pl.ds(col, 16)] = jnp.where(valid, dv * dv, 0.0)
            fidx_v[row, pl.ds(col, 16)] = jnp.where(
                valid, jvec * (N_NODES + 1), _MFLAT - 1)

        handles = []
        for j in range(_ROWS + 2):
            handles.append(
                pltpu.async_copy(norm_v.at[j], m_sh.at[fidx_v.at[j]],
                                 sem, add=True))
        for h in handles:
            h.wait()

        pltpu.sync_copy(m_sh, out_hbm)


@jax.jit
def _build_operator(edge_index):
    edges2 = edge_index.reshape(2 * _ROWS, 128)
    lut = lax.rsqrt(jnp.arange(1, _LUT + 1, dtype=jnp.float32))
    zeros = jnp.zeros((_MFLAT,), jnp.float32)
    ones = jnp.ones((_ROWS, 128), jnp.float32)
    mesh = plsc.VectorSubcoreMesh(core_axis_name="c", subcore_axis_name="s")
    m_flat = pl.kernel(
        _sc_body,
        out_type=jax.ShapeDtypeStruct((_MFLAT,), jnp.float32),
        mesh=mesh,
        compiler_params=pltpu.CompilerParams(needs_layout_passes=False),
        scratch_types=[
            pltpu.VMEM((2 * _ROWS, 128), jnp.int32),
            pltpu.VMEM((_LUT,), jnp.float32),
            pltpu.VMEM((_ROWS, 128), jnp.float32),
            pltpu.VMEM((_NPAD,), jnp.float32),
            pltpu.VMEM((_NPAD,), jnp.float32),
            pltpu.VMEM((_ROWS + 2, 128), jnp.float32),
            pltpu.VMEM((_ROWS + 2, 128), jnp.int32),
            pltpu.VMEM_SHARED((_MFLAT,), jnp.float32),
            pltpu.VMEM_SHARED((_NPAD,), jnp.float32),
            pltpu.SemaphoreType.DMA,
        ],
    )(edges2, lut, zeros, ones)
    return m_flat.reshape(N_NODES, N_NODES)


def _tc_body(params_ref, m_ref, w2_ref, b2_ref, w3_ref, b3_ref, x_ref,
             out_ref):
    b1 = params_ref[1]
    s2 = params_ref[2]
    bb = params_ref[3]
    xv = x_ref[...]
    h = lax.dot_general(xv, m_ref[...], (((1,), (0,)), ((), ())),
                        preferred_element_type=jnp.float32)
    h = jnp.tanh(h + b1) * s2 + bb
    h = lax.dot_general(h, w2_ref[...], (((1,), (1,)), ((), ())),
                        preferred_element_type=jnp.float32) + b2_ref[...]
    h = jnp.tanh(h)
    out_ref[...] = lax.dot_general(h, w3_ref[...], (((1,), (1,)), ((), ())),
                                   preferred_element_type=jnp.float32) + b3_ref[...]


def _fused_mlp(params, m, w2, b2, w3, b3, x2, block_b=1024):
    batch = x2.shape[0]
    n = x2.shape[1]
    d2 = w2.shape[0]
    d3 = w3.shape[0]
    grid = (batch // block_b,)
    return pl.pallas_call(
        _tc_body,
        grid=grid,
        in_specs=[
            pl.BlockSpec(memory_space=pltpu.SMEM),
            pl.BlockSpec((n, n), lambda i: (0, 0)),
            pl.BlockSpec((d2, n), lambda i: (0, 0)),
            pl.BlockSpec((1, d2), lambda i: (0, 0)),
            pl.BlockSpec((d3, d2), lambda i: (0, 0)),
            pl.BlockSpec((1, d3), lambda i: (0, 0)),
            pl.BlockSpec((block_b, n), lambda i: (i, 0)),
        ],
        out_specs=pl.BlockSpec((block_b, d3), lambda i: (i, 0)),
        out_shape=jax.ShapeDtypeStruct((batch, d3), jnp.float32),
        compiler_params=pltpu.CompilerParams(
            dimension_semantics=("arbitrary",)),
    )(params, m, w2, b2, w3, b3, x2)


def kernel(x, edge_index, gcn_w, gcn_b, bn_gamma, bn_beta, w2, b2, w3, b3):
    batch = x.shape[0]
    m = _build_operator(edge_index) * gcn_w[0, 0]
    params = jnp.stack([
        jnp.float32(1.0),
        gcn_b[0],
        bn_gamma[0] * lax.rsqrt(jnp.float32(1.0 + 1e-5)),
        bn_beta[0],
    ])
    x2 = x.reshape(batch, N_NODES)
    out = _fused_mlp(params, m, w2, b2.reshape(1, -1), w3,
                     b3.reshape(1, -1), x2)
    one = 1.0 + 0.0 * gcn_b[0]
    return (x * one, out.reshape(batch, 1, w3.shape[0]))
